# baseline (speedup 1.0000x reference)
import jax
import jax.numpy as jnp
from jax import lax
from jax.experimental import pallas as pl
from jax.experimental.pallas import tpu as pltpu

_B = 128
_d = 32
_D = 65536
_C = 2048
_NC = _D // _C
_GW = 2
_NG = _NC // _GW
_S = 8
_SPW = 0.1


def _mp_loss_body(x_ref, dictT_ref, dp_ref, out_ref, h1_ref, h2_ref, h3_ref):
    _SB = 2048

    def split_blk(i, _):
        blk = dp_ref[pl.ds(i * _SB, _SB), :]
        b1 = blk.astype(jnp.bfloat16)
        r1 = blk - b1.astype(jnp.float32)
        b2 = r1.astype(jnp.bfloat16)
        b3 = (r1 - b2.astype(jnp.float32)).astype(jnp.bfloat16)
        h1_ref[pl.ds(i * _SB, _SB), :] = b1
        h2_ref[pl.ds(i * _SB, _SB), :] = b2
        h3_ref[pl.ds(i * _SB, _SB), :] = b3
        return 0

    lax.fori_loop(0, (_D // 4) // _SB, split_blk, 0)

    x0 = x_ref[:]
    res = x0
    col_iota = lax.broadcasted_iota(jnp.int32, (_B, _C), 1)
    col_iota4 = lax.broadcasted_iota(jnp.int32, (_B, _C // 4), 1)
    grp_iota = lax.broadcasted_iota(jnp.int32, (_B, 4 * _d), 1) // _d

    idxs = []
    drows = []
    res_last = None
    val_last = None

    for step in range(_S):
        res_s = res

        def group_dots(g, res_s=res_s):
            def dot1(c):
                dT_c = dictT_ref[:, pl.ds(c * _C, _C)]
                return lax.dot_general(res_s, dT_c, (((1,), (0,)), ((), ())),
                                       preferred_element_type=jnp.float32)
            return tuple(dot1(g + k * _NG) for k in range(_GW))

        def one_chunk(c, ip):
            a = jnp.abs(ip)
            m = jnp.max(a, axis=1, keepdims=True)
            lidx = jnp.min(jnp.where(a == m, col_iota, _C),
                           axis=1, keepdims=True)
            oh4 = (col_iota4 == lax.shift_right_logical(lidx, 2)
                   ).astype(jnp.bfloat16)
            dn = (((1,), (0,)), ((), ()))
            base = c * (_C // 4)
            lpack = (
                lax.dot_general(oh4, h1_ref[pl.ds(base, _C // 4), :], dn,
                                preferred_element_type=jnp.float32)
                + lax.dot_general(oh4, h2_ref[pl.ds(base, _C // 4), :], dn,
                                  preferred_element_type=jnp.float32)
                + lax.dot_general(oh4, h3_ref[pl.ds(base, _C // 4), :], dn,
                                  preferred_element_type=jnp.float32))
            return (m, lidx + c * _C, lpack)

        def merge(k1, k2):
            u = k2[0] > k1[0]
            return tuple(jnp.where(u, b, a) for a, b in zip(k1, k2))

        def chunk_body(g, carry):
            run_m, run_i, run_p, ips = carry
            ips_next = group_dots(jnp.minimum(g + 1, _NG - 1))
            ks = [one_chunk(g + k * _NG, ips[k]) for k in range(_GW)]
            while len(ks) > 1:
                ks = [merge(ks[i], ks[i + 1]) for i in range(0, len(ks), 2)]
            m1, i1, p1 = ks[0]
            upd = (m1 > run_m) | ((m1 == run_m) & (i1 < run_i))
            return (jnp.where(upd, m1, run_m),
                    jnp.where(upd, i1, run_i),
                    jnp.where(upd, p1, run_p),
                    ips_next)

        init = (jnp.full((_B, 1), -1.0, jnp.float32),
                jnp.full((_B, 1), _D, jnp.int32),
                jnp.zeros((_B, 4 * _d), jnp.float32),
                group_dots(0))
        g_m, g_idx, g_pack, _ = lax.fori_loop(0, _NG, chunk_body, init)
        msk = jnp.where(grp_iota == (g_idx & 3), g_pack, 0.0)
        g_row = (msk[:, 0:_d] + msk[:, _d:2 * _d]
                 + msk[:, 2 * _d:3 * _d] + msk[:, 3 * _d:4 * _d])
        sgn = jnp.sum(res_s * g_row, axis=1, keepdims=True)
        g_val = jnp.where(sgn >= 0, g_m, -g_m)

        idxs.append(g_idx)
        drows.append(g_row)
        if step == _S - 1:
            res_last = res_s
            val_last = g_val
        res = res_s - g_val * g_row

    cs = [jnp.sum(res_last * drows[k], axis=1, keepdims=True)
          for k in range(_S - 1)]
    cs.append(val_last)

    recon = jnp.zeros((_B, _d), jnp.float32)
    l1 = jnp.zeros((_B, 1), jnp.float32)
    for k in range(_S):
        dup = jnp.zeros((_B, 1), dtype=jnp.bool_)
        for j in range(k):
            dup = jnp.logical_or(dup, idxs[k] == idxs[j])
        mk = jnp.where(dup, 0.0, 1.0)
        recon = recon + (mk * cs[k]) * drows[k]
        l1 = l1 + mk * jnp.abs(cs[k])

    diff = recon - x0
    rl = jnp.sum(jnp.sum(diff * diff, axis=1, keepdims=True),
                 axis=0, keepdims=True) / (_B * _d)
    sl = jnp.sum(l1, axis=0, keepdims=True) / _B
    out_ref[:, :] = rl + _SPW * sl


def kernel(x, dictionary):
    dictT = dictionary.T
    pk = (dictionary.shape[0] // 4, 4 * dictionary.shape[1])
    out = pl.pallas_call(
        _mp_loss_body,
        out_shape=jax.ShapeDtypeStruct((1, 1), jnp.float32),
        scratch_shapes=[pltpu.VMEM(pk, jnp.bfloat16)] * 3,
    )(x, dictT, dictionary.reshape(pk))
    return out[0, 0]


# speedup vs baseline: 1.0745x; 1.0745x over previous
import jax
import jax.numpy as jnp
from jax import lax
from jax.experimental import pallas as pl
from jax.experimental.pallas import tpu as pltpu

_B = 128
_d = 32
_D = 65536
_C = 2048
_NC = _D // _C
_GW = 4
_NG = _NC // _GW
_S = 8
_SPW = 0.1


def _mp_loss_body(x_ref, dictT_ref, dp_ref, out_ref, h1_ref, h2_ref, h3_ref):
    _SB = 2048

    def split_blk(i, _):
        blk = dp_ref[pl.ds(i * _SB, _SB), :]
        b1 = blk.astype(jnp.bfloat16)
        r1 = blk - b1.astype(jnp.float32)
        b2 = r1.astype(jnp.bfloat16)
        b3 = (r1 - b2.astype(jnp.float32)).astype(jnp.bfloat16)
        h1_ref[pl.ds(i * _SB, _SB), :] = b1
        h2_ref[pl.ds(i * _SB, _SB), :] = b2
        h3_ref[pl.ds(i * _SB, _SB), :] = b3
        return 0

    lax.fori_loop(0, (_D // 4) // _SB, split_blk, 0)

    x0 = x_ref[:]
    res = x0
    col_iota = lax.broadcasted_iota(jnp.int32, (_B, _C), 1)
    col_iota4 = lax.broadcasted_iota(jnp.int32, (_B, _C // 4), 1)
    grp_iota = lax.broadcasted_iota(jnp.int32, (_B, 4 * _d), 1) // _d

    idxs = []
    drows = []
    res_last = None
    val_last = None

    for step in range(_S):
        res_s = res

        def group_dots(g, res_s=res_s):
            def dot1(c):
                dT_c = dictT_ref[:, pl.ds(c * _C, _C)]
                return lax.dot_general(res_s, dT_c, (((1,), (0,)), ((), ())),
                                       preferred_element_type=jnp.float32)
            return tuple(dot1(g + k * _NG) for k in range(_GW))

        def one_chunk(c, ip):
            a = jnp.abs(ip)
            m = jnp.max(a, axis=1, keepdims=True)
            lidx = jnp.min(jnp.where(a == m, col_iota, _C),
                           axis=1, keepdims=True)
            oh4 = (col_iota4 == lax.shift_right_logical(lidx, 2)
                   ).astype(jnp.bfloat16)
            dn = (((1,), (0,)), ((), ()))
            base = c * (_C // 4)
            lpack = (
                lax.dot_general(oh4, h1_ref[pl.ds(base, _C // 4), :], dn,
                                preferred_element_type=jnp.float32)
                + lax.dot_general(oh4, h2_ref[pl.ds(base, _C // 4), :], dn,
                                  preferred_element_type=jnp.float32)
                + lax.dot_general(oh4, h3_ref[pl.ds(base, _C // 4), :], dn,
                                  preferred_element_type=jnp.float32))
            return (m, lidx + c * _C, lpack)

        def merge(k1, k2):
            u = k2[0] > k1[0]
            return tuple(jnp.where(u, b, a) for a, b in zip(k1, k2))

        def chunk_body(g, carry):
            run_m, run_i, run_p, ips = carry
            ips_next = group_dots(jnp.minimum(g + 1, _NG - 1))
            ks = [one_chunk(g + k * _NG, ips[k]) for k in range(_GW)]
            while len(ks) > 1:
                ks = [merge(ks[i], ks[i + 1]) for i in range(0, len(ks), 2)]
            m1, i1, p1 = ks[0]
            upd = (m1 > run_m) | ((m1 == run_m) & (i1 < run_i))
            return (jnp.where(upd, m1, run_m),
                    jnp.where(upd, i1, run_i),
                    jnp.where(upd, p1, run_p),
                    ips_next)

        init = (jnp.full((_B, 1), -1.0, jnp.float32),
                jnp.full((_B, 1), _D, jnp.int32),
                jnp.zeros((_B, 4 * _d), jnp.float32),
                group_dots(0))
        g_m, g_idx, g_pack, _ = lax.fori_loop(0, _NG, chunk_body, init)
        msk = jnp.where(grp_iota == (g_idx & 3), g_pack, 0.0)
        g_row = (msk[:, 0:_d] + msk[:, _d:2 * _d]
                 + msk[:, 2 * _d:3 * _d] + msk[:, 3 * _d:4 * _d])
        sgn = jnp.sum(res_s * g_row, axis=1, keepdims=True)
        g_val = jnp.where(sgn >= 0, g_m, -g_m)

        idxs.append(g_idx)
        drows.append(g_row)
        if step == _S - 1:
            res_last = res_s
            val_last = g_val
        res = res_s - g_val * g_row

    cs = [jnp.sum(res_last * drows[k], axis=1, keepdims=True)
          for k in range(_S - 1)]
    cs.append(val_last)

    recon = jnp.zeros((_B, _d), jnp.float32)
    l1 = jnp.zeros((_B, 1), jnp.float32)
    for k in range(_S):
        dup = jnp.zeros((_B, 1), dtype=jnp.bool_)
        for j in range(k):
            dup = jnp.logical_or(dup, idxs[k] == idxs[j])
        mk = jnp.where(dup, 0.0, 1.0)
        recon = recon + (mk * cs[k]) * drows[k]
        l1 = l1 + mk * jnp.abs(cs[k])

    diff = recon - x0
    rl = jnp.sum(jnp.sum(diff * diff, axis=1, keepdims=True),
                 axis=0, keepdims=True) / (_B * _d)
    sl = jnp.sum(l1, axis=0, keepdims=True) / _B
    out_ref[:, :] = rl + _SPW * sl


def kernel(x, dictionary):
    dictT = dictionary.T
    pk = (dictionary.shape[0] // 4, 4 * dictionary.shape[1])
    out = pl.pallas_call(
        _mp_loss_body,
        out_shape=jax.ShapeDtypeStruct((1, 1), jnp.float32),
        scratch_shapes=[pltpu.VMEM(pk, jnp.bfloat16)] * 3,
    )(x, dictT, dictionary.reshape(pk))
    return out[0, 0]
